# Initial kernel scaffold; baseline (speedup 1.0000x reference)
import jax
import jax.numpy as jnp
from jax.experimental import pallas as pl


def kernel(x, edge_index, W1, b1, W2, b2):
    raise NotImplementedError("write your pallas kernel here")


# speedup vs baseline: 26.2645x; 26.2645x over previous
import functools

import jax
import jax.numpy as jnp
from jax import lax
from jax.experimental import pallas as pl
from jax.experimental.pallas import tpu as pltpu
from jax.experimental.pallas import tpu_sc as plsc

N = 10000
D = 128
E = 320000
NC, NS, L = 2, 16, 16
NW = NC * NS
NCH = 79
EPW = NCH * 128
EPAD = NW * EPW
NP = 10240
RPT = NP // NS
OPT = N // NS


def _mesh():
    return plsc.VectorSubcoreMesh(
        core_axis_name="c", subcore_axis_name="s", num_cores=NC, num_subcores=NS
    )


_SC_PARAMS = pltpu.CompilerParams(needs_layout_passes=False)


def _sc_hist_body(dst_hbm, parts_hbm, idx_v, part_v):
    c = lax.axis_index("c")
    s = lax.axis_index("s")
    w = s * NC + c
    pltpu.sync_copy(dst_hbm.at[w], idx_v)

    def zero(i, _):
        part_v[pl.ds(i * L, L)] = jnp.zeros((L,), jnp.float32)
        return _

    lax.fori_loop(0, NP // L, zero, None)
    ones = jnp.ones((L,), jnp.float32)

    def body(i, _):
        idx = idx_v[pl.ds(i * L, L)]
        plsc.addupdate_scatter(part_v, [idx], ones)
        return _

    lax.fori_loop(0, EPW // L, body, None)
    pltpu.sync_copy(part_v, parts_hbm.at[w])


_sc_hist = pl.kernel(
    _sc_hist_body,
    out_type=jax.ShapeDtypeStruct((NW, NP), jnp.float32),
    mesh=_mesh(),
    compiler_params=_SC_PARAMS,
    scratch_types=[
        pltpu.VMEM((EPW,), jnp.int32),
        pltpu.VMEM((NP,), jnp.float32),
    ],
)


def _sc_sagg_body(vals_hbm, src_hbm, dst_hbm, parts_hbm, vals_v, idxs_v, idxd_v, part_v):
    c = lax.axis_index("c")
    s = lax.axis_index("s")
    w = s * NC + c
    pltpu.sync_copy(vals_hbm, vals_v)
    pltpu.sync_copy(src_hbm.at[w], idxs_v)
    pltpu.sync_copy(dst_hbm.at[w], idxd_v)

    def zero(i, _):
        part_v[pl.ds(i * L, L)] = jnp.zeros((L,), jnp.float32)
        return _

    lax.fori_loop(0, NP // L, zero, None)

    def body(i, _):
        si = idxs_v[pl.ds(i * L, L)]
        di = idxd_v[pl.ds(i * L, L)]
        v = plsc.load_gather(vals_v, [si])
        plsc.addupdate_scatter(part_v, [di], v)
        return _

    lax.fori_loop(0, EPW // L, body, None)
    pltpu.sync_copy(part_v, parts_hbm.at[w])


_sc_sagg = pl.kernel(
    _sc_sagg_body,
    out_type=jax.ShapeDtypeStruct((NW, NP), jnp.float32),
    mesh=_mesh(),
    compiler_params=_SC_PARAMS,
    scratch_types=[
        pltpu.VMEM((N,), jnp.float32),
        pltpu.VMEM((EPW,), jnp.int32),
        pltpu.VMEM((EPW,), jnp.int32),
        pltpu.VMEM((NP,), jnp.float32),
    ],
)


def _sc_ragg_body(hs_hbm, src_hbm, dst_hbm, p_hbm, isrc_v, idst_v, rows_v, acc_sh):
    c = lax.axis_index("c")
    s = lax.axis_index("s")
    w = s * NC + c
    pltpu.sync_copy(src_hbm.at[w], isrc_v)
    pltpu.sync_copy(dst_hbm.at[w], idst_v)

    def zrow(i, _):
        for k in range(D // L):
            rows_v[i, pl.ds(k * L, L)] = jnp.zeros((L,), jnp.float32)
        return _

    lax.fori_loop(0, 128, zrow, None)
    for j in range(RPT // 128):
        pltpu.sync_copy(rows_v, acc_sh.at[pl.ds(s * RPT + j * 128, 128)])
    plsc.subcore_barrier()

    def body(i, _):
        pltpu.sync_copy(hs_hbm.at[isrc_v.at[i]], rows_v)
        pltpu.sync_copy(rows_v, acc_sh.at[idst_v.at[i]], add=True)
        return _

    lax.fori_loop(0, NCH, body, None)
    plsc.subcore_barrier()
    pltpu.sync_copy(acc_sh.at[pl.ds(s * RPT, RPT)], p_hbm.at[c].at[pl.ds(s * RPT, RPT)])


_sc_ragg = pl.kernel(
    _sc_ragg_body,
    out_type=jax.ShapeDtypeStruct((NC, NP, D), jnp.float32),
    mesh=_mesh(),
    compiler_params=_SC_PARAMS,
    scratch_types=[
        pltpu.VMEM((NCH, 128), jnp.int32),
        pltpu.VMEM((NCH, 128), jnp.int32),
        pltpu.VMEM((128, D), jnp.float32),
        pltpu.VMEM_SHARED((NP, D), jnp.float32),
    ],
)


def _tc_dinv_body(parts_ref, dinv_ref):
    deg = jnp.sum(parts_ref[...], axis=0) + 1.0
    dinv_ref[...] = lax.rsqrt(deg)


def _tc_mm1_body(x_ref, w_ref, dinv_ref, hs_ref):
    h = jnp.dot(x_ref[...], w_ref[...], preferred_element_type=jnp.float32)
    hs_ref[...] = h * dinv_ref[0:N][:, None]


def _tc_mm2_body(p_ref, hs_ref, dinv_ref, b1_ref, w2_ref, gs_ref):
    dv = dinv_ref[0:N]
    agg = p_ref[0, 0:N] + p_ref[1, 0:N] + hs_ref[...]
    r = jnp.maximum(agg * dv[:, None] + b1_ref[...], 0.0)
    gs_ref[...] = jnp.sum(r * w2_ref[:, 0], axis=1) * dv


def _tc_fin_body(parts_ref, gs_ref, dinv_ref, b2_ref, out_ref):
    agg = jnp.sum(parts_ref[...], axis=0)[0:N] + gs_ref[...]
    out_ref[...] = (agg * dinv_ref[0:N] + b2_ref[0])[:, None]


def kernel(x, edge_index, W1, b1, W2, b2):
    src = edge_index[0].astype(jnp.int32)
    dst = edge_index[1].astype(jnp.int32)
    pad = EPAD - E
    srcp = jnp.concatenate([src, jnp.zeros((pad,), jnp.int32)])
    dstp = jnp.concatenate([dst, jnp.full((pad,), N, jnp.int32)])
    src_f = srcp.reshape(NW, EPW)
    dst_f = dstp.reshape(NW, EPW)
    src_c = srcp.reshape(NW, NCH, 128)
    dst_c = dstp.reshape(NW, NCH, 128)

    deg_parts = _sc_hist(dst_f)
    dinv = pl.pallas_call(
        _tc_dinv_body, out_shape=jax.ShapeDtypeStruct((NP,), jnp.float32)
    )(deg_parts)
    hs = pl.pallas_call(
        _tc_mm1_body, out_shape=jax.ShapeDtypeStruct((N, D), jnp.float32)
    )(x, W1, dinv)
    P = _sc_ragg(hs, src_c, dst_c)
    gs = pl.pallas_call(
        _tc_mm2_body, out_shape=jax.ShapeDtypeStruct((N,), jnp.float32)
    )(P, hs, dinv, b1, W2)
    parts2 = _sc_sagg(gs, src_f, dst_f)
    out = pl.pallas_call(
        _tc_fin_body, out_shape=jax.ShapeDtypeStruct((N, 1), jnp.float32)
    )(parts2, gs, dinv, b2)
    return out
